# baseline (speedup 1.0000x reference)
import functools

import jax
import jax.numpy as jnp
from jax import lax
from jax.experimental import pallas as pl
from jax.experimental.pallas import tpu as pltpu
from jax.experimental.pallas import tpu_sc as plsc

_NC = 2
_NS = 16


def kernel(indices, weight, trigger_embeds):
    B, L = indices.shape
    V, D = weight.shape
    NT = trigger_embeds.shape[0]
    NW = _NC * _NS
    assert B % NW == 0, (B, NW)
    seq_per_w = B // NW
    LG = L - NT

    idx = indices[:, NT:].astype(jnp.int32)

    mesh = plsc.VectorSubcoreMesh(core_axis_name="c", subcore_axis_name="s")

    @functools.partial(
        pl.kernel,
        out_type=jax.ShapeDtypeStruct((B, L, D), jnp.float32),
        mesh=mesh,
        scratch_types=[
            pltpu.VMEM((seq_per_w, LG), jnp.int32),
            pltpu.VMEM((NT, D), jnp.float32),
            pltpu.VMEM((LG, D), jnp.float32),
            pltpu.SemaphoreType.DMA,
            pltpu.SemaphoreType.DMA,
            pltpu.SemaphoreType.DMA,
        ],
        compiler_params=pltpu.CompilerParams(use_tc_tiling_on_sc=False),
    )
    def emb_kernel(idx_hbm, w_hbm, trig_hbm, out_hbm,
                   idx_v, trig_v, buf, gsem, ssem, tsem):
        wid = lax.axis_index("s") * _NC + lax.axis_index("c")
        seq0 = wid * seq_per_w
        pltpu.sync_copy(idx_hbm.at[pl.ds(seq0, seq_per_w)], idx_v)
        pltpu.sync_copy(trig_hbm, trig_v)

        @pl.loop(0, seq_per_w)
        def _per_seq(s):
            seq = seq0 + s
            pltpu.async_copy(trig_v, out_hbm.at[seq, pl.ds(0, NT)], tsem)
            pltpu.async_copy(w_hbm.at[idx_v.at[s]], buf, gsem).wait()
            pltpu.async_copy(buf, out_hbm.at[seq, pl.ds(NT, LG)], ssem).wait()

        @pl.loop(0, seq_per_w)
        def _drain_trig(s):
            pltpu.make_async_copy(trig_v, out_hbm.at[0, pl.ds(0, NT)], tsem).wait()

    return emb_kernel(idx, weight, trigger_embeds)


# speedup vs baseline: 1.0284x; 1.0284x over previous
import functools

import jax
import jax.numpy as jnp
from jax import lax
from jax.experimental import pallas as pl
from jax.experimental.pallas import tpu as pltpu
from jax.experimental.pallas import tpu_sc as plsc

_NC = 2
_NS = 16


def kernel(indices, weight, trigger_embeds):
    B, L = indices.shape
    V, D = weight.shape
    NT = trigger_embeds.shape[0]
    NW = _NC * _NS
    assert B % NW == 0, (B, NW)
    seq_per_w = B // NW
    LG = L - NT

    idx = indices[:, NT:].astype(jnp.int32)

    mesh = plsc.VectorSubcoreMesh(core_axis_name="c", subcore_axis_name="s")

    @functools.partial(
        pl.kernel,
        out_type=jax.ShapeDtypeStruct((B, L, D), jnp.float32),
        mesh=mesh,
        scratch_types=[
            pltpu.VMEM((seq_per_w, LG), jnp.int32),
            pltpu.VMEM((NT, D), jnp.float32),
            pltpu.VMEM((LG, D), jnp.float32),
            pltpu.VMEM((LG, D), jnp.float32),
            pltpu.SemaphoreType.DMA,
            pltpu.SemaphoreType.DMA,
            pltpu.SemaphoreType.DMA,
            pltpu.SemaphoreType.DMA,
            pltpu.SemaphoreType.DMA,
        ],
        compiler_params=pltpu.CompilerParams(use_tc_tiling_on_sc=False),
    )
    def emb_kernel(idx_hbm, w_hbm, trig_hbm, out_hbm,
                   idx_v, trig_v, buf0, buf1, g0, g1, s0, s1, tsem):
        bufs, gsems, ssems = [buf0, buf1], [g0, g1], [s0, s1]
        wid = lax.axis_index("s") * _NC + lax.axis_index("c")
        seq0 = wid * seq_per_w
        pltpu.sync_copy(idx_hbm.at[pl.ds(seq0, seq_per_w)], idx_v)
        pltpu.sync_copy(trig_hbm, trig_v)

        pltpu.async_copy(w_hbm.at[idx_v.at[0]], buf0, g0)

        half = seq_per_w // 2

        @pl.loop(0, half)
        def _pair(go):
            for i in range(2):
                s = go * 2 + i
                buf, gsem, ssem = bufs[i], gsems[i], ssems[i]
                bufp, gsemp, ssemp = bufs[1 - i], gsems[1 - i], ssems[1 - i]

                pltpu.make_async_copy(w_hbm.at[pl.ds(0, LG)], buf, gsem).wait()

                def _wait_prev_scatter(ref=bufp, sem=ssemp):
                    pltpu.make_async_copy(
                        ref, out_hbm.at[0, pl.ds(NT, LG)], sem).wait()
                if i == 1:
                    _wait_prev_scatter()
                else:
                    pl.when(go >= 1)(_wait_prev_scatter)

                def _start_next_gather(sv=s, ref=bufp, sem=gsemp):
                    pltpu.async_copy(w_hbm.at[idx_v.at[sv + 1]], ref, sem)
                if i == 0:
                    _start_next_gather()
                else:
                    pl.when(go <= half - 2)(_start_next_gather)

                pltpu.async_copy(buf, out_hbm.at[seq0 + s, pl.ds(NT, LG)], ssem)
                pltpu.async_copy(trig_v, out_hbm.at[seq0 + s, pl.ds(0, NT)], tsem)

        pltpu.make_async_copy(
            bufs[1], out_hbm.at[0, pl.ds(NT, LG)], ssems[1]).wait()

        @pl.loop(0, seq_per_w)
        def _drain_trig(s):
            pltpu.make_async_copy(trig_v, out_hbm.at[0, pl.ds(0, NT)], tsem).wait()

    return emb_kernel(idx, weight, trigger_embeds)
